# baseline (speedup 1.0000x reference)
import functools

import jax
import jax.numpy as jnp
from jax import lax
from jax.experimental import pallas as pl
from jax.experimental.pallas import tpu as pltpu
from jax.experimental.pallas import tpu_sc as plsc

N = 10000
E = 320000
HID = 64
IN = 128

NC = 2
NS = 16
NW = NC * NS
ROWS_PER_W = 80
CHUNK = 8
ROWS = NW * ROWS_PER_W
EPAD = ROWS * 128
NPAD = 10112
RPT = NPAD // NS

_MESH = plsc.VectorSubcoreMesh(core_axis_name="c", subcore_axis_name="s")


@functools.partial(
    pl.kernel,
    out_type=jax.ShapeDtypeStruct((NC, NPAD, HID), jnp.float32),
    mesh=_MESH,
    scratch_types=[
        pltpu.VMEM_SHARED((NPAD, HID), jnp.float32),
        pltpu.VMEM((CHUNK, 128), jnp.int32),
        pltpu.VMEM((CHUNK, 128), jnp.int32),
        pltpu.VMEM((CHUNK, 128, HID), jnp.float32),
        pltpu.SemaphoreType.DMA,
    ],
    compiler_params=pltpu.CompilerParams(use_tc_tiling_on_sc=False),
)
def _sc_propagate(u_hbm, src_hbm, dst_hbm, zero_hbm, out_hbm,
                  acc, sbuf, dbuf, gbuf, sem):
    c = lax.axis_index("c")
    s = lax.axis_index("s")
    pltpu.sync_copy(zero_hbm, acc.at[pl.ds(s * RPT, RPT)])
    plsc.subcore_barrier()

    row0 = (c * NS + s) * ROWS_PER_W

    def body(i, carry):
        base = row0 + i * CHUNK
        pltpu.sync_copy(src_hbm.at[pl.ds(base, CHUNK)], sbuf)
        pltpu.sync_copy(dst_hbm.at[pl.ds(base, CHUNK)], dbuf)
        copies = [
            pltpu.async_copy(u_hbm.at[sbuf.at[j]], gbuf.at[j], sem)
            for j in range(CHUNK)
        ]
        for cp in copies:
            cp.wait()
        for j in range(CHUNK):
            pltpu.sync_copy(gbuf.at[j], acc.at[dbuf.at[j]], add=True)
        return carry

    lax.fori_loop(0, ROWS_PER_W // CHUNK, body, 0)
    plsc.subcore_barrier()
    pltpu.sync_copy(acc.at[pl.ds(s * RPT, RPT)],
                    out_hbm.at[c, pl.ds(s * RPT, RPT)])


@functools.partial(
    pl.kernel,
    out_type=jax.ShapeDtypeStruct((NC, NPAD, 8), jnp.float32),
    mesh=_MESH,
    scratch_types=[
        pltpu.VMEM_SHARED((NPAD, 8), jnp.float32),
        pltpu.VMEM((CHUNK, 128), jnp.int32),
        pltpu.VMEM((128, 8), jnp.float32),
    ],
    compiler_params=pltpu.CompilerParams(use_tc_tiling_on_sc=False),
)
def _sc_degree(dst_hbm, ones_hbm, zero_hbm, out_hbm, acc, dbuf, obuf):
    c = lax.axis_index("c")
    s = lax.axis_index("s")
    pltpu.sync_copy(zero_hbm, acc.at[pl.ds(s * RPT, RPT)])
    pltpu.sync_copy(ones_hbm, obuf)
    plsc.subcore_barrier()

    row0 = (c * NS + s) * ROWS_PER_W

    def body(i, carry):
        base = row0 + i * CHUNK
        pltpu.sync_copy(dst_hbm.at[pl.ds(base, CHUNK)], dbuf)
        for j in range(CHUNK):
            pltpu.sync_copy(obuf, acc.at[dbuf.at[j]], add=True)
        return carry

    lax.fori_loop(0, ROWS_PER_W // CHUNK, body, 0)
    plsc.subcore_barrier()
    pltpu.sync_copy(acc.at[pl.ds(s * RPT, RPT)],
                    out_hbm.at[c, pl.ds(s * RPT, RPT)])


_BM = 1000


def _row_spec(d):
    return pl.BlockSpec((_BM, d), lambda i: (i, 0))


def _full_spec(r, d):
    return pl.BlockSpec((r, d), lambda i: (0, 0))


def _tc_call(body, in_specs, out_dim, n_out=1):
    if n_out == 1:
        out_shape = jax.ShapeDtypeStruct((N, out_dim), jnp.float32)
        out_specs = _row_spec(out_dim)
    else:
        out_shape = [jax.ShapeDtypeStruct((N, out_dim), jnp.float32)] * n_out
        out_specs = [_row_spec(out_dim)] * n_out
    return pl.pallas_call(
        body,
        grid=(N // _BM,),
        in_specs=in_specs,
        out_specs=out_specs,
        out_shape=out_shape,
    )


def _k_xw1(x_ref, w_ref, o_ref):
    o_ref[...] = jnp.dot(x_ref[...], w_ref[...],
                         preferred_element_type=jnp.float32)


def _k_dinv_u1(p0_ref, p1_ref, t1_ref, dinv_ref, u1_ref):
    deg = p0_ref[:, 0:1] + p1_ref[:, 0:1] + 1.0
    dinv = lax.rsqrt(jnp.broadcast_to(deg, (_BM, HID)))
    dinv_ref[...] = dinv
    u1_ref[...] = t1_ref[...] * dinv


def _k_mid(s0_ref, s1_ref, u_ref, dinv_ref, b_ref, w_ref, o_ref):
    dinv = dinv_ref[...]
    h = dinv * (s0_ref[...] + s1_ref[...] + u_ref[...]) + b_ref[...]
    h = jnp.maximum(h, 0.0)
    o_ref[...] = jnp.dot(h, w_ref[...],
                         preferred_element_type=jnp.float32) * dinv


def _k_emb(s0_ref, s1_ref, u_ref, dinv_ref, b_ref, o_ref):
    dinv = dinv_ref[...]
    emb = dinv * (s0_ref[...] + s1_ref[...] + u_ref[...]) + b_ref[...]
    o_ref[...] = emb * dinv


def _k_out(s0_ref, s1_ref, u_ref, dinv_ref, w_ref, b_ref, o_ref):
    ph = dinv_ref[...] * (s0_ref[...] + s1_ref[...] + u_ref[...])
    o_ref[...] = jnp.dot(ph, w_ref[...],
                         preferred_element_type=jnp.float32) + b_ref[...]


def kernel(x, edge_index, W1, b1, W2, b2, W3, b3, W4, b4):
    ei = edge_index.astype(jnp.int32)
    pad = EPAD - E
    srcp = jnp.concatenate([ei[0], jnp.zeros((pad,), jnp.int32)]).reshape(ROWS, 128)
    pad_dst = N + (jnp.arange(pad, dtype=jnp.int32) % (NPAD - N))
    dstp = jnp.concatenate([ei[1], pad_dst]).reshape(ROWS, 128)
    zeros_h = jnp.zeros((RPT, HID), jnp.float32)
    zeros_8 = jnp.zeros((RPT, 8), jnp.float32)
    ones_8 = jnp.ones((128, 8), jnp.float32)

    degp = _sc_degree(dstp, ones_8, zeros_8)
    t1 = _tc_call(_k_xw1, [_row_spec(IN), _full_spec(IN, HID)], HID)(x, W1)

    dinv, u1 = _tc_call(
        _k_dinv_u1, [_row_spec(8), _row_spec(8), _row_spec(HID)], HID, n_out=2,
    )(degp[0], degp[1], t1)

    sp = _sc_propagate(u1, srcp, dstp, zeros_h)
    u2 = _tc_call(
        _k_mid,
        [_row_spec(HID)] * 4 + [_full_spec(1, HID), _full_spec(HID, HID)],
        HID,
    )(sp[0], sp[1], u1, dinv, b1.reshape(1, HID), W2)

    sp = _sc_propagate(u2, srcp, dstp, zeros_h)
    u3 = _tc_call(
        _k_mid,
        [_row_spec(HID)] * 4 + [_full_spec(1, HID), _full_spec(HID, HID)],
        HID,
    )(sp[0], sp[1], u2, dinv, b2.reshape(1, HID), W3)

    sp = _sc_propagate(u3, srcp, dstp, zeros_h)
    u4 = _tc_call(
        _k_emb, [_row_spec(HID)] * 4 + [_full_spec(1, HID)], HID,
    )(sp[0], sp[1], u3, dinv, b3.reshape(1, HID))

    sp = _sc_propagate(u4, srcp, dstp, zeros_h)
    x_ = _tc_call(
        _k_out,
        [_row_spec(HID)] * 4 + [_full_spec(HID, IN), _full_spec(1, IN)],
        IN,
    )(sp[0], sp[1], u4, dinv, W4, b4.reshape(1, IN))
    return x_


# speedup vs baseline: 1.0682x; 1.0682x over previous
import functools

import jax
import jax.numpy as jnp
from jax import lax
from jax.experimental import pallas as pl
from jax.experimental.pallas import tpu as pltpu
from jax.experimental.pallas import tpu_sc as plsc

N = 10000
E = 320000
HID = 64
IN = 128

NC = 2
NS = 16
NW = NC * NS
RW0 = 112
RW1 = 48
CHUNK = 8
ROWS = NS * (RW0 + RW1)
ROWS_PER_W = ROWS // NW
EPAD = ROWS * 128
NPAD = 10112
RPT = NPAD // NS

_MESH = plsc.VectorSubcoreMesh(core_axis_name="c", subcore_axis_name="s")


@functools.partial(
    pl.kernel,
    out_type=jax.ShapeDtypeStruct((NC, NPAD, HID), jnp.float32),
    mesh=_MESH,
    scratch_types=[
        pltpu.VMEM_SHARED((NPAD, HID), jnp.float32),
        pltpu.VMEM((CHUNK, 128), jnp.int32),
        pltpu.VMEM((CHUNK, 128), jnp.int32),
        pltpu.VMEM((CHUNK, 128, HID), jnp.float32),
        pltpu.SemaphoreType.DMA,
    ],
    compiler_params=pltpu.CompilerParams(use_tc_tiling_on_sc=False),
)
def _sc_propagate(u_hbm, src_hbm, dst_hbm, zero_hbm, out_hbm,
                  acc, sbuf, dbuf, gbuf, sem):
    c = lax.axis_index("c")
    s = lax.axis_index("s")
    pltpu.sync_copy(zero_hbm, acc.at[pl.ds(s * RPT, RPT)])
    plsc.subcore_barrier()

    row0 = lax.select(c == 0, s * RW0, NS * RW0 + s * RW1)
    n_chunks = lax.select(c == 0, RW0 // CHUNK, RW1 // CHUNK)

    def body(i, carry):
        base = row0 + i * CHUNK
        pltpu.sync_copy(src_hbm.at[pl.ds(base, CHUNK)], sbuf)
        pltpu.sync_copy(dst_hbm.at[pl.ds(base, CHUNK)], dbuf)
        copies = [
            pltpu.async_copy(u_hbm.at[sbuf.at[j]], gbuf.at[j], sem)
            for j in range(CHUNK)
        ]
        for cp in copies:
            cp.wait()
        for j in range(CHUNK):
            pltpu.sync_copy(gbuf.at[j], acc.at[dbuf.at[j]], add=True)
        return carry

    lax.fori_loop(0, n_chunks, body, 0)
    plsc.subcore_barrier()
    pltpu.sync_copy(acc.at[pl.ds(s * RPT, RPT)],
                    out_hbm.at[c, pl.ds(s * RPT, RPT)])


@functools.partial(
    pl.kernel,
    out_type=jax.ShapeDtypeStruct((NC, NPAD, 8), jnp.float32),
    mesh=_MESH,
    scratch_types=[
        pltpu.VMEM_SHARED((NPAD, 8), jnp.float32),
        pltpu.VMEM((CHUNK, 128), jnp.int32),
        pltpu.VMEM((128, 8), jnp.float32),
    ],
    compiler_params=pltpu.CompilerParams(use_tc_tiling_on_sc=False),
)
def _sc_degree(dst_hbm, ones_hbm, zero_hbm, out_hbm, acc, dbuf, obuf):
    c = lax.axis_index("c")
    s = lax.axis_index("s")
    pltpu.sync_copy(zero_hbm, acc.at[pl.ds(s * RPT, RPT)])
    pltpu.sync_copy(ones_hbm, obuf)
    plsc.subcore_barrier()

    row0 = (c * NS + s) * ROWS_PER_W

    def body(i, carry):
        base = row0 + i * CHUNK
        pltpu.sync_copy(dst_hbm.at[pl.ds(base, CHUNK)], dbuf)
        for j in range(CHUNK):
            pltpu.sync_copy(obuf, acc.at[dbuf.at[j]], add=True)
        return carry

    lax.fori_loop(0, ROWS_PER_W // CHUNK, body, 0)
    plsc.subcore_barrier()
    pltpu.sync_copy(acc.at[pl.ds(s * RPT, RPT)],
                    out_hbm.at[c, pl.ds(s * RPT, RPT)])


_BM = 1000


def _row_spec(d):
    return pl.BlockSpec((_BM, d), lambda i: (i, 0))


def _full_spec(r, d):
    return pl.BlockSpec((r, d), lambda i: (0, 0))


def _tc_call(body, in_specs, out_dim, n_out=1):
    if n_out == 1:
        out_shape = jax.ShapeDtypeStruct((N, out_dim), jnp.float32)
        out_specs = _row_spec(out_dim)
    else:
        out_shape = [jax.ShapeDtypeStruct((N, out_dim), jnp.float32)] * n_out
        out_specs = [_row_spec(out_dim)] * n_out
    return pl.pallas_call(
        body,
        grid=(N // _BM,),
        in_specs=in_specs,
        out_specs=out_specs,
        out_shape=out_shape,
    )


def _k_xw1(x_ref, w_ref, o_ref):
    o_ref[...] = jnp.dot(x_ref[...], w_ref[...],
                         preferred_element_type=jnp.float32)


def _k_dinv_u1(p0_ref, p1_ref, t1_ref, dinv_ref, u1_ref):
    deg = p0_ref[:, 0:1] + p1_ref[:, 0:1] + 1.0
    dinv = lax.rsqrt(jnp.broadcast_to(deg, (_BM, HID)))
    dinv_ref[...] = dinv
    u1_ref[...] = t1_ref[...] * dinv


def _k_mid(s0_ref, s1_ref, u_ref, dinv_ref, b_ref, w_ref, o_ref):
    dinv = dinv_ref[...]
    h = dinv * (s0_ref[...] + s1_ref[...] + u_ref[...]) + b_ref[...]
    h = jnp.maximum(h, 0.0)
    o_ref[...] = jnp.dot(h, w_ref[...],
                         preferred_element_type=jnp.float32) * dinv


def _k_emb(s0_ref, s1_ref, u_ref, dinv_ref, b_ref, o_ref):
    dinv = dinv_ref[...]
    emb = dinv * (s0_ref[...] + s1_ref[...] + u_ref[...]) + b_ref[...]
    o_ref[...] = emb * dinv


def _k_out(s0_ref, s1_ref, u_ref, dinv_ref, w_ref, b_ref, o_ref):
    ph = dinv_ref[...] * (s0_ref[...] + s1_ref[...] + u_ref[...])
    o_ref[...] = jnp.dot(ph, w_ref[...],
                         preferred_element_type=jnp.float32) + b_ref[...]


def kernel(x, edge_index, W1, b1, W2, b2, W3, b3, W4, b4):
    ei = edge_index.astype(jnp.int32)
    pad = EPAD - E
    srcp = jnp.concatenate([ei[0], jnp.zeros((pad,), jnp.int32)]).reshape(ROWS, 128)
    pad_dst = N + (jnp.arange(pad, dtype=jnp.int32) % (NPAD - N))
    dstp = jnp.concatenate([ei[1], pad_dst]).reshape(ROWS, 128)
    zeros_h = jnp.zeros((RPT, HID), jnp.float32)
    zeros_8 = jnp.zeros((RPT, 8), jnp.float32)
    ones_8 = jnp.ones((128, 8), jnp.float32)

    degp = _sc_degree(dstp, ones_8, zeros_8)
    t1 = _tc_call(_k_xw1, [_row_spec(IN), _full_spec(IN, HID)], HID)(x, W1)

    dinv, u1 = _tc_call(
        _k_dinv_u1, [_row_spec(8), _row_spec(8), _row_spec(HID)], HID, n_out=2,
    )(degp[0], degp[1], t1)

    sp = _sc_propagate(u1, srcp, dstp, zeros_h)
    u2 = _tc_call(
        _k_mid,
        [_row_spec(HID)] * 4 + [_full_spec(1, HID), _full_spec(HID, HID)],
        HID,
    )(sp[0], sp[1], u1, dinv, b1.reshape(1, HID), W2)

    sp = _sc_propagate(u2, srcp, dstp, zeros_h)
    u3 = _tc_call(
        _k_mid,
        [_row_spec(HID)] * 4 + [_full_spec(1, HID), _full_spec(HID, HID)],
        HID,
    )(sp[0], sp[1], u2, dinv, b2.reshape(1, HID), W3)

    sp = _sc_propagate(u3, srcp, dstp, zeros_h)
    u4 = _tc_call(
        _k_emb, [_row_spec(HID)] * 4 + [_full_spec(1, HID)], HID,
    )(sp[0], sp[1], u3, dinv, b3.reshape(1, HID))

    sp = _sc_propagate(u4, srcp, dstp, zeros_h)
    x_ = _tc_call(
        _k_out,
        [_row_spec(HID)] * 4 + [_full_spec(HID, IN), _full_spec(1, IN)],
        IN,
    )(sp[0], sp[1], u4, dinv, W4, b4.reshape(1, IN))
    return x_


# speedup vs baseline: 1.1982x; 1.1217x over previous
import functools

import jax
import jax.numpy as jnp
from jax import lax
from jax.experimental import pallas as pl
from jax.experimental.pallas import tpu as pltpu
from jax.experimental.pallas import tpu_sc as plsc

N = 10000
E = 320000
HID = 64
IN = 128

NC = 2
NS = 16
NW = NC * NS
RW0 = 136
RW1 = 24
CHUNK = 8
ROWS = NS * (RW0 + RW1)
ROWS_PER_W = ROWS // NW
EPAD = ROWS * 128
NPAD = 10112
RPT = NPAD // NS

_MESH = plsc.VectorSubcoreMesh(core_axis_name="c", subcore_axis_name="s")


@functools.partial(
    pl.kernel,
    out_type=jax.ShapeDtypeStruct((NC, NPAD, HID), jnp.float32),
    mesh=_MESH,
    scratch_types=[
        pltpu.VMEM_SHARED((NPAD, HID), jnp.float32),
        pltpu.VMEM((CHUNK, 128), jnp.int32),
        pltpu.VMEM((CHUNK, 128), jnp.int32),
        pltpu.VMEM((CHUNK, 128, HID), jnp.float32),
        pltpu.SemaphoreType.DMA,
    ],
    compiler_params=pltpu.CompilerParams(use_tc_tiling_on_sc=False),
)
def _sc_propagate(u_hbm, src_hbm, dst_hbm, zero_hbm, out_hbm,
                  acc, sbuf, dbuf, gbuf, sem):
    c = lax.axis_index("c")
    s = lax.axis_index("s")
    pltpu.sync_copy(zero_hbm, acc.at[pl.ds(s * RPT, RPT)])
    plsc.subcore_barrier()

    row0 = lax.select(c == 0, s * RW0, NS * RW0 + s * RW1)
    n_chunks = lax.select(c == 0, RW0 // CHUNK, RW1 // CHUNK)

    def body(i, carry):
        base = row0 + i * CHUNK
        pltpu.sync_copy(src_hbm.at[pl.ds(base, CHUNK)], sbuf)
        pltpu.sync_copy(dst_hbm.at[pl.ds(base, CHUNK)], dbuf)
        copies = [
            pltpu.async_copy(u_hbm.at[sbuf.at[j]], gbuf.at[j], sem)
            for j in range(CHUNK)
        ]
        for cp in copies:
            cp.wait()
        for j in range(CHUNK):
            pltpu.sync_copy(gbuf.at[j], acc.at[dbuf.at[j]], add=True)
        return carry

    lax.fori_loop(0, n_chunks, body, 0)
    plsc.subcore_barrier()
    pltpu.sync_copy(acc.at[pl.ds(s * RPT, RPT)],
                    out_hbm.at[c, pl.ds(s * RPT, RPT)])


@functools.partial(
    pl.kernel,
    out_type=jax.ShapeDtypeStruct((NC, NPAD, 8), jnp.float32),
    mesh=_MESH,
    scratch_types=[
        pltpu.VMEM_SHARED((NPAD, 8), jnp.float32),
        pltpu.VMEM((CHUNK, 128), jnp.int32),
        pltpu.VMEM((128, 8), jnp.float32),
    ],
    compiler_params=pltpu.CompilerParams(use_tc_tiling_on_sc=False),
)
def _sc_degree(dst_hbm, ones_hbm, zero_hbm, out_hbm, acc, dbuf, obuf):
    c = lax.axis_index("c")
    s = lax.axis_index("s")
    pltpu.sync_copy(zero_hbm, acc.at[pl.ds(s * RPT, RPT)])
    pltpu.sync_copy(ones_hbm, obuf)
    plsc.subcore_barrier()

    row0 = (c * NS + s) * ROWS_PER_W

    def body(i, carry):
        base = row0 + i * CHUNK
        pltpu.sync_copy(dst_hbm.at[pl.ds(base, CHUNK)], dbuf)
        for j in range(CHUNK):
            pltpu.sync_copy(obuf, acc.at[dbuf.at[j]], add=True)
        return carry

    lax.fori_loop(0, ROWS_PER_W // CHUNK, body, 0)
    plsc.subcore_barrier()
    pltpu.sync_copy(acc.at[pl.ds(s * RPT, RPT)],
                    out_hbm.at[c, pl.ds(s * RPT, RPT)])


_BM = 1000


def _row_spec(d):
    return pl.BlockSpec((_BM, d), lambda i: (i, 0))


def _full_spec(r, d):
    return pl.BlockSpec((r, d), lambda i: (0, 0))


def _tc_call(body, in_specs, out_dim, n_out=1):
    if n_out == 1:
        out_shape = jax.ShapeDtypeStruct((N, out_dim), jnp.float32)
        out_specs = _row_spec(out_dim)
    else:
        out_shape = [jax.ShapeDtypeStruct((N, out_dim), jnp.float32)] * n_out
        out_specs = [_row_spec(out_dim)] * n_out
    return pl.pallas_call(
        body,
        grid=(N // _BM,),
        in_specs=in_specs,
        out_specs=out_specs,
        out_shape=out_shape,
    )


def _k_xw1(x_ref, w_ref, o_ref):
    o_ref[...] = jnp.dot(x_ref[...], w_ref[...],
                         preferred_element_type=jnp.float32)


def _k_dinv_u1(p0_ref, p1_ref, t1_ref, dinv_ref, u1_ref):
    deg = p0_ref[:, 0:1] + p1_ref[:, 0:1] + 1.0
    dinv = lax.rsqrt(jnp.broadcast_to(deg, (_BM, HID)))
    dinv_ref[...] = dinv
    u1_ref[...] = t1_ref[...] * dinv


def _k_mid(s0_ref, s1_ref, u_ref, dinv_ref, b_ref, w_ref, o_ref):
    dinv = dinv_ref[...]
    h = dinv * (s0_ref[...] + s1_ref[...] + u_ref[...]) + b_ref[...]
    h = jnp.maximum(h, 0.0)
    o_ref[...] = jnp.dot(h, w_ref[...],
                         preferred_element_type=jnp.float32) * dinv


def _k_emb(s0_ref, s1_ref, u_ref, dinv_ref, b_ref, o_ref):
    dinv = dinv_ref[...]
    emb = dinv * (s0_ref[...] + s1_ref[...] + u_ref[...]) + b_ref[...]
    o_ref[...] = emb * dinv


def _k_out(s0_ref, s1_ref, u_ref, dinv_ref, w_ref, b_ref, o_ref):
    ph = dinv_ref[...] * (s0_ref[...] + s1_ref[...] + u_ref[...])
    o_ref[...] = jnp.dot(ph, w_ref[...],
                         preferred_element_type=jnp.float32) + b_ref[...]


def kernel(x, edge_index, W1, b1, W2, b2, W3, b3, W4, b4):
    ei = edge_index.astype(jnp.int32)
    pad = EPAD - E
    srcp = jnp.concatenate([ei[0], jnp.zeros((pad,), jnp.int32)]).reshape(ROWS, 128)
    pad_dst = N + (jnp.arange(pad, dtype=jnp.int32) % (NPAD - N))
    dstp = jnp.concatenate([ei[1], pad_dst]).reshape(ROWS, 128)
    zeros_h = jnp.zeros((RPT, HID), jnp.float32)
    zeros_8 = jnp.zeros((RPT, 8), jnp.float32)
    ones_8 = jnp.ones((128, 8), jnp.float32)

    degp = _sc_degree(dstp, ones_8, zeros_8)
    t1 = _tc_call(_k_xw1, [_row_spec(IN), _full_spec(IN, HID)], HID)(x, W1)

    dinv, u1 = _tc_call(
        _k_dinv_u1, [_row_spec(8), _row_spec(8), _row_spec(HID)], HID, n_out=2,
    )(degp[0], degp[1], t1)

    sp = _sc_propagate(u1, srcp, dstp, zeros_h)
    u2 = _tc_call(
        _k_mid,
        [_row_spec(HID)] * 4 + [_full_spec(1, HID), _full_spec(HID, HID)],
        HID,
    )(sp[0], sp[1], u1, dinv, b1.reshape(1, HID), W2)

    sp = _sc_propagate(u2, srcp, dstp, zeros_h)
    u3 = _tc_call(
        _k_mid,
        [_row_spec(HID)] * 4 + [_full_spec(1, HID), _full_spec(HID, HID)],
        HID,
    )(sp[0], sp[1], u2, dinv, b2.reshape(1, HID), W3)

    sp = _sc_propagate(u3, srcp, dstp, zeros_h)
    u4 = _tc_call(
        _k_emb, [_row_spec(HID)] * 4 + [_full_spec(1, HID)], HID,
    )(sp[0], sp[1], u3, dinv, b3.reshape(1, HID))

    sp = _sc_propagate(u4, srcp, dstp, zeros_h)
    x_ = _tc_call(
        _k_out,
        [_row_spec(HID)] * 4 + [_full_spec(HID, IN), _full_spec(1, IN)],
        IN,
    )(sp[0], sp[1], u4, dinv, W4, b4.reshape(1, IN))
    return x_
